# Initial kernel scaffold; baseline (speedup 1.0000x reference)
import jax
import jax.numpy as jnp
from jax.experimental import pallas as pl


def kernel(inp, edgidx, h, Wxz, Whz, Wxr, Whr, Wxh, Whh):
    raise NotImplementedError("write your pallas kernel here")


# speedup vs baseline: 9.0411x; 9.0411x over previous
import functools

import jax
import jax.numpy as jnp
from jax import lax
from jax.experimental import pallas as pl
from jax.experimental.pallas import tpu as pltpu
from jax.experimental.pallas import tpu_sc as plsc

N = 10000
E = 320000
D = 128
NC = 2
NS = 16
K = 80
STRIPE = 624
STRIPE_LAST = N - STRIPE * (NS - 1)
BK = 1000
GRID = N // BK

@functools.lru_cache
def _get_mesh():
    return plsc.VectorSubcoreMesh(core_axis_name="c", subcore_axis_name="s",
                                  num_cores=NC, num_subcores=NS)


def _stripes(s, fn):
    @pl.when(s < NS - 1)
    def _():
        fn(s * STRIPE, STRIPE)

    @pl.when(s == NS - 1)
    def _():
        fn(STRIPE * (NS - 1), STRIPE_LAST)


def _edge_loop(table, esrc, edst, acc, srcv, dstv, rows, sem, start, n_chunks):
    def chunk(j, carry):
        off = start + j * K
        pltpu.sync_copy(esrc.at[pl.ds(off, K)], srcv)
        pltpu.sync_copy(edst.at[pl.ds(off, K)], dstv)
        pltpu.async_copy(table.at[srcv], rows, sem).wait()
        pltpu.sync_copy(rows, acc.at[dstv], add=True)
        return carry

    lax.fori_loop(0, n_chunks, chunk, 0, unroll=False)


def _rows_body(epc, ya, yb, esrc, edst, zeros, out, srcv, dstv, rows, acc, sem):
    c = lax.axis_index("c")
    s = lax.axis_index("s")
    _stripes(s, lambda off, sz: pltpu.sync_copy(
        zeros.at[pl.ds(off, sz)], acc.at[pl.ds(off, sz)]))
    plsc.subcore_barrier()

    ept = epc // NS
    n_chunks = ept // K
    tile_start = c * (E - epc) + s * ept

    @pl.when(c == 0)
    def _():
        _edge_loop(ya, esrc, edst, acc, srcv, dstv, rows, sem, tile_start, n_chunks)

    @pl.when(c == 1)
    def _():
        _edge_loop(yb, esrc, edst, acc, srcv, dstv, rows, sem, tile_start, n_chunks)

    plsc.subcore_barrier()
    _stripes(s, lambda off, sz: pltpu.sync_copy(
        acc.at[pl.ds(off, sz)], out.at[c, pl.ds(off, sz)]))


@functools.lru_cache
def _make_rows_pass(epc):
    return functools.partial(
        pl.kernel,
        functools.partial(_rows_body, epc),
        out_type=jax.ShapeDtypeStruct((NC, N, D), jnp.float32),
        mesh=_get_mesh(),
        scratch_types=[
            pltpu.VMEM((K,), jnp.int32),
            pltpu.VMEM((K,), jnp.int32),
            pltpu.VMEM((K, D), jnp.float32),
            pltpu.VMEM_SHARED((N, D), jnp.float32),
            pltpu.SemaphoreType.DMA,
        ],
    )()

DEGW = 128


def _deg_body_w(degw, edst, zeros16, ones16, out, dstv, onesv, acc, sem):
    c = lax.axis_index("c")
    s = lax.axis_index("s")
    _stripes(s, lambda off, sz: pltpu.sync_copy(
        zeros16.at[pl.ds(off, sz)], acc.at[pl.ds(off, sz)]))
    pltpu.sync_copy(ones16, onesv)
    plsc.subcore_barrier()

    ept = (E // NC) // NS
    n_chunks = ept // K
    tile_start = c * (E // NC) + s * ept

    def chunk(j, carry):
        off = tile_start + j * K
        pltpu.sync_copy(edst.at[pl.ds(off, K)], dstv)
        pltpu.sync_copy(onesv, acc.at[dstv], add=True)
        return carry

    lax.fori_loop(0, n_chunks, chunk, 0, unroll=False)

    plsc.subcore_barrier()
    _stripes(s, lambda off, sz: pltpu.sync_copy(
        acc.at[pl.ds(off, sz)], out.at[c, pl.ds(off, sz)]))


@functools.lru_cache
def _get_deg_pass():
    return functools.partial(
        pl.kernel,
        functools.partial(_deg_body_w, DEGW),
        out_type=jax.ShapeDtypeStruct((NC, N, DEGW), jnp.float32),
        mesh=_get_mesh(),
        scratch_types=[
            pltpu.VMEM((K,), jnp.int32),
            pltpu.VMEM((K, DEGW), jnp.float32),
            pltpu.VMEM_SHARED((N, DEGW), jnp.float32),
            pltpu.SemaphoreType.DMA,
        ],
    )()


_row_spec = pl.BlockSpec((BK, D), lambda i: (i, 0))
_w_spec = pl.BlockSpec((D, D), lambda i: (0, 0))
_deg_spec = pl.BlockSpec((BK, DEGW), lambda i: (i, 0))


def _mm(a, w):
    return jnp.dot(a, w, preferred_element_type=jnp.float32)


def _prep_body(dga, dgb, inp, h0, h1, u_o, yx0_o, yh0_o, yh1_o):
    deg = dga[...] + dgb[...] + 2.0
    u = lax.rsqrt(deg[:, 0:1])
    u_o[...] = jnp.broadcast_to(u, (BK, D))
    yx0_o[...] = u * inp[...]
    yh0_o[...] = u * h0[...]
    yh1_o[...] = u * h1[...]


def _prep(dga, dgb, inp, h0, h1):
    f32 = jnp.float32
    return pl.pallas_call(
        _prep_body,
        grid=(GRID,),
        in_specs=[_deg_spec, _deg_spec, _row_spec, _row_spec, _row_spec],
        out_specs=[_row_spec, _row_spec, _row_spec, _row_spec],
        out_shape=[jax.ShapeDtypeStruct((N, D), f32)] * 4,
    )(dga, dgb, inp, h0, h1)


def _stage1_body(nparts, *refs):
    refs = list(refs)
    tx = refs.pop(0)[...]
    if nparts == 2:
        tx = tx + refs.pop(0)[...]
    th, xin, h, u = (refs.pop(0)[...] for _ in range(4))
    wxz, whz, wxr, whr, wxh = (refs.pop(0)[...] for _ in range(5))
    z_o, gx_o, yrh_o = refs
    cx = u * (tx + 2.0 * u * xin)
    ch = u * (th + 2.0 * u * h)
    relu = lambda v: jnp.maximum(v, 0.0)
    z = jax.nn.sigmoid(relu(_mm(cx, wxz)) + relu(_mm(ch, whz)))
    r = jax.nn.sigmoid(relu(_mm(cx, wxr)) + relu(_mm(ch, whr)))
    z_o[...] = z
    gx_o[...] = relu(_mm(cx, wxh))
    yrh_o[...] = u * (r * h)


def _stage1(tx_parts, th, xin, h, u, wxz, whz, wxr, whr, wxh):
    f32 = jnp.float32
    nparts = len(tx_parts)
    return pl.pallas_call(
        functools.partial(_stage1_body, nparts),
        grid=(GRID,),
        in_specs=[_row_spec] * (nparts + 4) + [_w_spec] * 5,
        out_specs=[_row_spec] * 3,
        out_shape=[jax.ShapeDtypeStruct((N, D), f32)] * 3,
    )(*tx_parts, th, xin, h, u, wxz, whz, wxr, whr, wxh)


def _stage2_body(nparts, emit_ynext, *refs):
    refs = list(refs)
    trh = refs.pop(0)[...]
    if nparts == 2:
        trh = trh + refs.pop(0)[...]
    yrh, u, z, gx, h = (refs.pop(0)[...] for _ in range(5))
    whh = refs.pop(0)[...]
    crh = u * (trh + 2.0 * yrh)
    ht = jnp.tanh(gx + jnp.maximum(_mm(crh, whh), 0.0))
    hout = z * h + (1.0 - z) * ht
    refs[0][...] = hout
    if emit_ynext:
        refs[1][...] = u * hout


def _stage2(trh_parts, yrh, u, z, gx, h, whh, emit_ynext):
    f32 = jnp.float32
    nparts = len(trh_parts)
    n_out = 2 if emit_ynext else 1
    return pl.pallas_call(
        functools.partial(_stage2_body, nparts, emit_ynext),
        grid=(GRID,),
        in_specs=[_row_spec] * (nparts + 5) + [_w_spec],
        out_specs=[_row_spec] * n_out,
        out_shape=[jax.ShapeDtypeStruct((N, D), f32)] * n_out,
    )(*trh_parts, yrh, u, z, gx, h, whh)


def kernel(inp, edgidx, h, Wxz, Whz, Wxr, Whr, Wxh, Whh):
    f32 = jnp.float32
    esrc = edgidx[0].astype(jnp.int32)
    edst = edgidx[1].astype(jnp.int32)
    zeros = jnp.zeros((N, D), f32)
    zeros16 = jnp.zeros((N, DEGW), f32)
    ones16 = jnp.ones((K, DEGW), f32)

    rows_paired = _make_rows_pass(E)
    rows_split = _make_rows_pass(E // 2)

    dg = _get_deg_pass()(edst, zeros16, ones16)
    u, yx0, yh0, yh1 = _prep(dg[0], dg[1], inp, h[0], h[1])

    p1 = rows_paired(yx0, yh0, esrc, edst, zeros)
    z0, gx0, yrh0 = _stage1([p1[0]], p1[1], inp, h[0], u,
                            Wxz[0], Whz[0], Wxr[0], Whr[0], Wxh[0])
    p2 = rows_paired(yrh0, yh1, esrc, edst, zeros)
    hout0, yx1 = _stage2([p2[0]], yrh0, u, z0, gx0, h[0], Whh[0], True)

    p3 = rows_split(yx1, yx1, esrc, edst, zeros)
    z1, gx1, yrh1 = _stage1([p3[0], p3[1]], p2[1], hout0, h[1], u,
                            Wxz[1], Whz[1], Wxr[1], Whr[1], Wxh[1])
    p4 = rows_split(yrh1, yrh1, esrc, edst, zeros)
    (hout1,) = _stage2([p4[0], p4[1]], yrh1, u, z1, gx1, h[1], Whh[1], False)

    h_out = jnp.stack([hout0, hout1], axis=0)
    return (h_out, h_out)


# speedup vs baseline: 19.3304x; 2.1381x over previous
import functools

import jax
import jax.numpy as jnp
from jax import lax
from jax.experimental import pallas as pl
from jax.experimental.pallas import tpu as pltpu
from jax.experimental.pallas import tpu_sc as plsc

N = 10000
E = 320000
D = 128
NC = 2
NS = 16
K = 125
CH = 8
ROWS_TOTAL = E // K
STRIPE = 624
STRIPE_LAST = N - STRIPE * (NS - 1)
BK = 1000
GRID = N // BK

@functools.lru_cache
def _get_mesh():
    return plsc.VectorSubcoreMesh(core_axis_name="c", subcore_axis_name="s",
                                  num_cores=NC, num_subcores=NS)


def _stripes(s, fn):
    @pl.when(s < NS - 1)
    def _():
        fn(s * STRIPE, STRIPE)

    @pl.when(s == NS - 1)
    def _():
        fn(STRIPE * (NS - 1), STRIPE_LAST)


def _edge_loop(table, esrc2, edst2, acc, srcv, dstv, rows, gsems, ssems,
               row0, n_super):
    def gfire(j, p):
        return pltpu.async_copy(table.at[srcv.at[j]], rows.at[p], gsems[p])

    def sfire(j, p):
        pltpu.async_copy(rows.at[p], acc.at[dstv.at[j]], ssems[p], add=True)

    def sdrain(j, p):
        pltpu.make_async_copy(rows.at[p], acc.at[dstv.at[j]], ssems[p]).wait()

    def body(t, carry):
        @pl.when(t > 0)
        def _():
            sdrain(CH - 2, 0)
            sdrain(CH - 1, 1)

        roff = row0 + t * CH
        pltpu.sync_copy(esrc2.at[pl.ds(roff, CH)], srcv)
        pltpu.sync_copy(edst2.at[pl.ds(roff, CH)], dstv)
        g = gfire(0, 0)
        for j in range(CH):
            p = j % 2
            if j + 1 < CH:
                if j > 0:
                    sdrain(j - 1, 1 - p)
                gn = gfire(j + 1, 1 - p)
            g.wait()
            sfire(j, p)
            if j + 1 < CH:
                g = gn
        return carry

    lax.fori_loop(0, n_super, body, 0, unroll=False)
    sdrain(CH - 2, 0)
    sdrain(CH - 1, 1)


def _rows_body(epc, ya, yb, esrc2, edst2, zeros, out, srcv, dstv, rows, acc,
               *sems):
    gsems, ssems = sems[:2], sems[2:]
    c = lax.axis_index("c")
    s = lax.axis_index("s")
    _stripes(s, lambda off, sz: pltpu.sync_copy(
        zeros.at[pl.ds(off, sz)], acc.at[pl.ds(off, sz)]))
    plsc.subcore_barrier()

    rows_per_core = epc // K
    rpt = rows_per_core // NS
    n_super = rpt // CH
    row0 = c * (ROWS_TOTAL - rows_per_core) + s * rpt

    @pl.when(c == 0)
    def _():
        _edge_loop(ya, esrc2, edst2, acc, srcv, dstv, rows, gsems, ssems,
                   row0, n_super)

    @pl.when(c == 1)
    def _():
        _edge_loop(yb, esrc2, edst2, acc, srcv, dstv, rows, gsems, ssems,
                   row0, n_super)

    plsc.subcore_barrier()
    _stripes(s, lambda off, sz: pltpu.sync_copy(
        acc.at[pl.ds(off, sz)], out.at[c, pl.ds(off, sz)]))


@functools.lru_cache
def _make_rows_pass(epc):
    return functools.partial(
        pl.kernel,
        functools.partial(_rows_body, epc),
        out_type=jax.ShapeDtypeStruct((NC, N, D), jnp.float32),
        mesh=_get_mesh(),
        scratch_types=[
            pltpu.VMEM((CH, K), jnp.int32),
            pltpu.VMEM((CH, K), jnp.int32),
            pltpu.VMEM((2, K, D), jnp.float32),
            pltpu.VMEM_SHARED((N, D), jnp.float32),
        ] + [pltpu.SemaphoreType.DMA] * 4,
    )()


def _deg_body(edst2, zeros, ones, out, dstv, onesv, acc, *ssems):
    c = lax.axis_index("c")
    s = lax.axis_index("s")
    _stripes(s, lambda off, sz: pltpu.sync_copy(
        zeros.at[pl.ds(off, sz)], acc.at[pl.ds(off, sz)]))
    pltpu.sync_copy(ones, onesv)
    plsc.subcore_barrier()

    rows_per_core = ROWS_TOTAL // NC
    rpt = rows_per_core // NS
    n_super = rpt // CH
    row0 = c * rows_per_core + s * rpt

    def drain():
        for b in range(CH):
            pltpu.make_async_copy(onesv, acc.at[dstv.at[b]], ssems[b]).wait()

    def body(t, carry):
        roff = row0 + t * CH

        @pl.when(t > 0)
        def _():
            drain()

        pltpu.sync_copy(edst2.at[pl.ds(roff, CH)], dstv)
        for b in range(CH):
            pltpu.async_copy(onesv, acc.at[dstv.at[b]], ssems[b], add=True)
        return carry

    lax.fori_loop(0, n_super, body, 0, unroll=False)
    drain()

    plsc.subcore_barrier()
    _stripes(s, lambda off, sz: pltpu.sync_copy(
        acc.at[pl.ds(off, sz)], out.at[c, pl.ds(off, sz)]))


@functools.lru_cache
def _get_deg_pass():
    return functools.partial(
        pl.kernel,
        _deg_body,
        out_type=jax.ShapeDtypeStruct((NC, N, D), jnp.float32),
        mesh=_get_mesh(),
        scratch_types=[
            pltpu.VMEM((CH, K), jnp.int32),
            pltpu.VMEM((K, D), jnp.float32),
            pltpu.VMEM_SHARED((N, D), jnp.float32),
        ] + [pltpu.SemaphoreType.DMA] * CH,
    )()


_row_spec = pl.BlockSpec((BK, D), lambda i: (i, 0))
_w_spec = pl.BlockSpec((D, D), lambda i: (0, 0))


def _mm(a, w):
    return jnp.dot(a, w, preferred_element_type=jnp.float32)


def _prep_body(dga, dgb, inp, h0, h1, u_o, yx0_o, yh0_o, yh1_o):
    u = lax.rsqrt(dga[...] + dgb[...] + 2.0)
    u_o[...] = u
    yx0_o[...] = u * inp[...]
    yh0_o[...] = u * h0[...]
    yh1_o[...] = u * h1[...]


def _prep(dga, dgb, inp, h0, h1):
    f32 = jnp.float32
    return pl.pallas_call(
        _prep_body,
        grid=(GRID,),
        in_specs=[_row_spec] * 5,
        out_specs=[_row_spec] * 4,
        out_shape=[jax.ShapeDtypeStruct((N, D), f32)] * 4,
    )(dga, dgb, inp, h0, h1)


def _stage1_body(nparts, *refs):
    refs = list(refs)
    tx = refs.pop(0)[...]
    if nparts == 2:
        tx = tx + refs.pop(0)[...]
    th, xin, h, u = (refs.pop(0)[...] for _ in range(4))
    wxz, whz, wxr, whr, wxh = (refs.pop(0)[...] for _ in range(5))
    z_o, gx_o, yrh_o = refs
    cx = u * (tx + 2.0 * u * xin)
    ch = u * (th + 2.0 * u * h)
    relu = lambda v: jnp.maximum(v, 0.0)
    z = jax.nn.sigmoid(relu(_mm(cx, wxz)) + relu(_mm(ch, whz)))
    r = jax.nn.sigmoid(relu(_mm(cx, wxr)) + relu(_mm(ch, whr)))
    z_o[...] = z
    gx_o[...] = relu(_mm(cx, wxh))
    yrh_o[...] = u * (r * h)


def _stage1(tx_parts, th, xin, h, u, wxz, whz, wxr, whr, wxh):
    f32 = jnp.float32
    nparts = len(tx_parts)
    return pl.pallas_call(
        functools.partial(_stage1_body, nparts),
        grid=(GRID,),
        in_specs=[_row_spec] * (nparts + 4) + [_w_spec] * 5,
        out_specs=[_row_spec] * 3,
        out_shape=[jax.ShapeDtypeStruct((N, D), f32)] * 3,
    )(*tx_parts, th, xin, h, u, wxz, whz, wxr, whr, wxh)


def _stage2_body(nparts, emit_ynext, *refs):
    refs = list(refs)
    trh = refs.pop(0)[...]
    if nparts == 2:
        trh = trh + refs.pop(0)[...]
    yrh, u, z, gx, h = (refs.pop(0)[...] for _ in range(5))
    whh = refs.pop(0)[...]
    crh = u * (trh + 2.0 * yrh)
    ht = jnp.tanh(gx + jnp.maximum(_mm(crh, whh), 0.0))
    hout = z * h + (1.0 - z) * ht
    refs[0][...] = hout
    if emit_ynext:
        refs[1][...] = u * hout


def _stage2(trh_parts, yrh, u, z, gx, h, whh, emit_ynext):
    f32 = jnp.float32
    nparts = len(trh_parts)
    n_out = 2 if emit_ynext else 1
    return pl.pallas_call(
        functools.partial(_stage2_body, nparts, emit_ynext),
        grid=(GRID,),
        in_specs=[_row_spec] * (nparts + 5) + [_w_spec],
        out_specs=[_row_spec] * n_out,
        out_shape=[jax.ShapeDtypeStruct((N, D), f32)] * n_out,
    )(*trh_parts, yrh, u, z, gx, h, whh)


def kernel(inp, edgidx, h, Wxz, Whz, Wxr, Whr, Wxh, Whh):
    f32 = jnp.float32
    esrc = edgidx[0].astype(jnp.int32).reshape(ROWS_TOTAL, K)
    edst = edgidx[1].astype(jnp.int32).reshape(ROWS_TOTAL, K)
    zeros = jnp.zeros((N, D), f32)
    ones = jnp.ones((K, D), f32)

    rows_paired = _make_rows_pass(E)
    rows_split = _make_rows_pass(E // 2)

    dg = _get_deg_pass()(edst, zeros, ones)
    u, yx0, yh0, yh1 = _prep(dg[0], dg[1], inp, h[0], h[1])

    p1 = rows_paired(yx0, yh0, esrc, edst, zeros)
    z0, gx0, yrh0 = _stage1([p1[0]], p1[1], inp, h[0], u,
                            Wxz[0], Whz[0], Wxr[0], Whr[0], Wxh[0])
    p2 = rows_paired(yrh0, yh1, esrc, edst, zeros)
    hout0, yx1 = _stage2([p2[0]], yrh0, u, z0, gx0, h[0], Whh[0], True)

    p3 = rows_split(yx1, yx1, esrc, edst, zeros)
    z1, gx1, yrh1 = _stage1([p3[0], p3[1]], p2[1], hout0, h[1], u,
                            Wxz[1], Whz[1], Wxr[1], Whr[1], Wxh[1])
    p4 = rows_split(yrh1, yrh1, esrc, edst, zeros)
    (hout1,) = _stage2([p4[0], p4[1]], yrh1, u, z1, gx1, h[1], Whh[1], False)

    h_out = jnp.stack([hout0, hout1], axis=0)
    return (h_out, h_out)


# speedup vs baseline: 20.8015x; 1.0761x over previous
import functools

import jax
import jax.numpy as jnp
from jax import lax
from jax.experimental import pallas as pl
from jax.experimental.pallas import tpu as pltpu
from jax.experimental.pallas import tpu_sc as plsc

N = 10000
E = 320000
D = 128
NC = 2
NS = 16
K = 125
CH = 16
ROWS_TOTAL = E // K
STRIPE = 624
STRIPE_LAST = N - STRIPE * (NS - 1)
BK = 1000
GRID = N // BK

@functools.lru_cache
def _get_mesh():
    return plsc.VectorSubcoreMesh(core_axis_name="c", subcore_axis_name="s",
                                  num_cores=NC, num_subcores=NS)


def _stripes(s, fn):
    @pl.when(s < NS - 1)
    def _():
        fn(s * STRIPE, STRIPE)

    @pl.when(s == NS - 1)
    def _():
        fn(STRIPE * (NS - 1), STRIPE_LAST)


def _edge_loop(table, esrc2, edst2, acc, srcv, dstv, rows, gsems, ssems,
               row0, n_super):
    def gfire(j, p):
        return pltpu.async_copy(table.at[srcv.at[j]], rows.at[p], gsems[p])

    def sfire(j, p):
        pltpu.async_copy(rows.at[p], acc.at[dstv.at[j]], ssems[p], add=True)

    def sdrain(j, p):
        pltpu.make_async_copy(rows.at[p], acc.at[dstv.at[j]], ssems[p]).wait()

    def body(t, carry):
        @pl.when(t > 0)
        def _():
            sdrain(CH - 2, 0)
            sdrain(CH - 1, 1)

        roff = row0 + t * CH
        pltpu.sync_copy(esrc2.at[pl.ds(roff, CH)], srcv)
        pltpu.sync_copy(edst2.at[pl.ds(roff, CH)], dstv)
        g = gfire(0, 0)
        for j in range(CH):
            p = j % 2
            if j + 1 < CH:
                if j > 0:
                    sdrain(j - 1, 1 - p)
                gn = gfire(j + 1, 1 - p)
            g.wait()
            sfire(j, p)
            if j + 1 < CH:
                g = gn
        return carry

    lax.fori_loop(0, n_super, body, 0, unroll=False)
    sdrain(CH - 2, 0)
    sdrain(CH - 1, 1)


def _rows_body(epc, ya, yb, esrc2, edst2, zeros, out, srcv, dstv, rows, acc,
               *sems):
    gsems, ssems = sems[:2], sems[2:]
    c = lax.axis_index("c")
    s = lax.axis_index("s")
    _stripes(s, lambda off, sz: pltpu.sync_copy(
        zeros.at[pl.ds(off, sz)], acc.at[pl.ds(off, sz)]))
    plsc.subcore_barrier()

    rows_per_core = epc // K
    rpt = rows_per_core // NS
    n_super = rpt // CH
    row0 = c * (ROWS_TOTAL - rows_per_core) + s * rpt

    @pl.when(c == 0)
    def _():
        _edge_loop(ya, esrc2, edst2, acc, srcv, dstv, rows, gsems, ssems,
                   row0, n_super)

    @pl.when(c == 1)
    def _():
        _edge_loop(yb, esrc2, edst2, acc, srcv, dstv, rows, gsems, ssems,
                   row0, n_super)

    plsc.subcore_barrier()
    _stripes(s, lambda off, sz: pltpu.sync_copy(
        acc.at[pl.ds(off, sz)], out.at[c, pl.ds(off, sz)]))


@functools.lru_cache
def _make_rows_pass(epc):
    return functools.partial(
        pl.kernel,
        functools.partial(_rows_body, epc),
        out_type=jax.ShapeDtypeStruct((NC, N, D), jnp.float32),
        mesh=_get_mesh(),
        scratch_types=[
            pltpu.VMEM((CH, K), jnp.int32),
            pltpu.VMEM((CH, K), jnp.int32),
            pltpu.VMEM((2, K, D), jnp.float32),
            pltpu.VMEM_SHARED((N, D), jnp.float32),
        ] + [pltpu.SemaphoreType.DMA] * 4,
    )()


def _deg_body(edst2, zeros, ones, out, dstv, onesv, acc, *ssems):
    c = lax.axis_index("c")
    s = lax.axis_index("s")
    _stripes(s, lambda off, sz: pltpu.sync_copy(
        zeros.at[pl.ds(off, sz)], acc.at[pl.ds(off, sz)]))
    pltpu.sync_copy(ones, onesv)
    plsc.subcore_barrier()

    rows_per_core = ROWS_TOTAL // NC
    rpt = rows_per_core // NS
    n_super = rpt // CH
    row0 = c * rows_per_core + s * rpt

    def drain():
        for b in range(CH):
            pltpu.make_async_copy(onesv, acc.at[dstv.at[b]], ssems[b]).wait()

    def body(t, carry):
        roff = row0 + t * CH

        @pl.when(t > 0)
        def _():
            drain()

        pltpu.sync_copy(edst2.at[pl.ds(roff, CH)], dstv)
        for b in range(CH):
            pltpu.async_copy(onesv, acc.at[dstv.at[b]], ssems[b], add=True)
        return carry

    lax.fori_loop(0, n_super, body, 0, unroll=False)
    drain()

    plsc.subcore_barrier()
    _stripes(s, lambda off, sz: pltpu.sync_copy(
        acc.at[pl.ds(off, sz)], out.at[c, pl.ds(off, sz)]))


@functools.lru_cache
def _get_deg_pass():
    return functools.partial(
        pl.kernel,
        _deg_body,
        out_type=jax.ShapeDtypeStruct((NC, N, D), jnp.float32),
        mesh=_get_mesh(),
        scratch_types=[
            pltpu.VMEM((CH, K), jnp.int32),
            pltpu.VMEM((K, D), jnp.float32),
            pltpu.VMEM_SHARED((N, D), jnp.float32),
        ] + [pltpu.SemaphoreType.DMA] * CH,
    )()


_row_spec = pl.BlockSpec((BK, D), lambda i: (i, 0))
_w_spec = pl.BlockSpec((D, D), lambda i: (0, 0))


def _mm(a, w):
    return jnp.dot(a, w, preferred_element_type=jnp.float32)


def _prep_body(dga, dgb, inp, h0, h1, u_o, yx0_o, yh0_o, yh1_o):
    u = lax.rsqrt(dga[...] + dgb[...] + 2.0)
    u_o[...] = u
    yx0_o[...] = u * inp[...]
    yh0_o[...] = u * h0[...]
    yh1_o[...] = u * h1[...]


def _prep(dga, dgb, inp, h0, h1):
    f32 = jnp.float32
    return pl.pallas_call(
        _prep_body,
        grid=(GRID,),
        in_specs=[_row_spec] * 5,
        out_specs=[_row_spec] * 4,
        out_shape=[jax.ShapeDtypeStruct((N, D), f32)] * 4,
    )(dga, dgb, inp, h0, h1)


def _stage1_body(nparts, *refs):
    refs = list(refs)
    tx = refs.pop(0)[...]
    if nparts == 2:
        tx = tx + refs.pop(0)[...]
    th, xin, h, u = (refs.pop(0)[...] for _ in range(4))
    wxz, whz, wxr, whr, wxh = (refs.pop(0)[...] for _ in range(5))
    z_o, gx_o, yrh_o = refs
    cx = u * (tx + 2.0 * u * xin)
    ch = u * (th + 2.0 * u * h)
    relu = lambda v: jnp.maximum(v, 0.0)
    z = jax.nn.sigmoid(relu(_mm(cx, wxz)) + relu(_mm(ch, whz)))
    r = jax.nn.sigmoid(relu(_mm(cx, wxr)) + relu(_mm(ch, whr)))
    z_o[...] = z
    gx_o[...] = relu(_mm(cx, wxh))
    yrh_o[...] = u * (r * h)


def _stage1(tx_parts, th, xin, h, u, wxz, whz, wxr, whr, wxh):
    f32 = jnp.float32
    nparts = len(tx_parts)
    return pl.pallas_call(
        functools.partial(_stage1_body, nparts),
        grid=(GRID,),
        in_specs=[_row_spec] * (nparts + 4) + [_w_spec] * 5,
        out_specs=[_row_spec] * 3,
        out_shape=[jax.ShapeDtypeStruct((N, D), f32)] * 3,
    )(*tx_parts, th, xin, h, u, wxz, whz, wxr, whr, wxh)


def _stage2_body(nparts, emit_ynext, *refs):
    refs = list(refs)
    trh = refs.pop(0)[...]
    if nparts == 2:
        trh = trh + refs.pop(0)[...]
    yrh, u, z, gx, h = (refs.pop(0)[...] for _ in range(5))
    whh = refs.pop(0)[...]
    crh = u * (trh + 2.0 * yrh)
    ht = jnp.tanh(gx + jnp.maximum(_mm(crh, whh), 0.0))
    hout = z * h + (1.0 - z) * ht
    refs[0][...] = hout
    if emit_ynext:
        refs[1][...] = u * hout


def _stage2(trh_parts, yrh, u, z, gx, h, whh, emit_ynext):
    f32 = jnp.float32
    nparts = len(trh_parts)
    n_out = 2 if emit_ynext else 1
    return pl.pallas_call(
        functools.partial(_stage2_body, nparts, emit_ynext),
        grid=(GRID,),
        in_specs=[_row_spec] * (nparts + 5) + [_w_spec],
        out_specs=[_row_spec] * n_out,
        out_shape=[jax.ShapeDtypeStruct((N, D), f32)] * n_out,
    )(*trh_parts, yrh, u, z, gx, h, whh)


def kernel(inp, edgidx, h, Wxz, Whz, Wxr, Whr, Wxh, Whh):
    f32 = jnp.float32
    esrc = edgidx[0].astype(jnp.int32).reshape(ROWS_TOTAL, K)
    edst = edgidx[1].astype(jnp.int32).reshape(ROWS_TOTAL, K)
    zeros = jnp.zeros((N, D), f32)
    ones = jnp.ones((K, D), f32)

    rows_paired = _make_rows_pass(E)
    rows_split = _make_rows_pass(E // 2)

    dg = _get_deg_pass()(edst, zeros, ones)
    u, yx0, yh0, yh1 = _prep(dg[0], dg[1], inp, h[0], h[1])

    p1 = rows_paired(yx0, yh0, esrc, edst, zeros)
    z0, gx0, yrh0 = _stage1([p1[0]], p1[1], inp, h[0], u,
                            Wxz[0], Whz[0], Wxr[0], Whr[0], Wxh[0])
    p2 = rows_paired(yrh0, yh1, esrc, edst, zeros)
    hout0, yx1 = _stage2([p2[0]], yrh0, u, z0, gx0, h[0], Whh[0], True)

    p3 = rows_split(yx1, yx1, esrc, edst, zeros)
    z1, gx1, yrh1 = _stage1([p3[0], p3[1]], p2[1], hout0, h[1], u,
                            Wxz[1], Whz[1], Wxr[1], Whr[1], Wxh[1])
    p4 = rows_split(yrh1, yrh1, esrc, edst, zeros)
    (hout1,) = _stage2([p4[0], p4[1]], yrh1, u, z1, gx1, h[1], Whh[1], False)

    h_out = jnp.stack([hout0, hout1], axis=0)
    return (h_out, h_out)


# speedup vs baseline: 21.5382x; 1.0354x over previous
import functools

import jax
import jax.numpy as jnp
from jax import lax
from jax.experimental import pallas as pl
from jax.experimental.pallas import tpu as pltpu
from jax.experimental.pallas import tpu_sc as plsc

N = 10000
E = 320000
D = 128
NC = 2
NS = 16
K = 125
CH = 16
ROWS_TOTAL = E // K
STRIPE = 624
STRIPE_LAST = N - STRIPE * (NS - 1)
BK = 1000
GRID = N // BK

@functools.lru_cache
def _get_mesh():
    return plsc.VectorSubcoreMesh(core_axis_name="c", subcore_axis_name="s",
                                  num_cores=NC, num_subcores=NS)


def _stripes(s, fn):
    @pl.when(s < NS - 1)
    def _():
        fn(s * STRIPE, STRIPE)

    @pl.when(s == NS - 1)
    def _():
        fn(STRIPE * (NS - 1), STRIPE_LAST)


def _edge_loop(table, esrc2, edst2, acc, srcv, dstv, rows, gsems, ssems, isems,
               row0, n_super):
    def gfire(q, j, p):
        return pltpu.async_copy(table.at[srcv.at[q, j]], rows.at[p], gsems[p])

    def sfire(q, j, p):
        pltpu.async_copy(rows.at[p], acc.at[dstv.at[q, j]], ssems[p], add=True)

    def sdrain(q, j, p):
        pltpu.make_async_copy(rows.at[p], acc.at[dstv.at[q, j]], ssems[p]).wait()

    def ifire(t, q):
        roff = row0 + t * CH
        pltpu.async_copy(esrc2.at[pl.ds(roff, CH)], srcv.at[q], isems[0])
        pltpu.async_copy(edst2.at[pl.ds(roff, CH)], dstv.at[q], isems[1])

    def iwait(q):
        pltpu.make_async_copy(esrc2.at[pl.ds(0, CH)], srcv.at[q], isems[0]).wait()
        pltpu.make_async_copy(edst2.at[pl.ds(0, CH)], dstv.at[q], isems[1]).wait()

    def chunks(q):
        g = gfire(q, 0, 0)
        for j in range(CH):
            p = j % 2
            if j + 1 < CH:
                if j > 0:
                    sdrain(q, j - 1, 1 - p)
                gn = gfire(q, j + 1, 1 - p)
            g.wait()
            sfire(q, j, p)
            if j + 1 < CH:
                g = gn

    def tdrain(q):
        sdrain(q, CH - 2, 0)
        sdrain(q, CH - 1, 1)

    ifire(0, 0)
    iwait(0)
    ifire(1, 1)
    chunks(0)

    pairs = (n_super - 1) // 2

    def body(m, carry):
        t1 = 1 + 2 * m
        tdrain(0)
        iwait(1)
        ifire(t1 + 1, 0)
        chunks(1)
        t2 = t1 + 1
        tdrain(1)
        iwait(0)

        @pl.when(t2 + 1 < n_super)
        def _():
            ifire(t2 + 1, 1)

        chunks(0)
        return carry

    lax.fori_loop(0, pairs, body, 0, unroll=False)

    if (n_super - 1) - 2 * pairs:
        tdrain(0)
        iwait(1)
        chunks(1)
        tdrain(1)
    else:
        tdrain(0)


def _rows_body(epc, ya, yb, esrc2, edst2, zeros, out, srcv, dstv, rows, acc,
               *sems):
    gsems, ssems, isems = sems[:2], sems[2:4], sems[4:]
    c = lax.axis_index("c")
    s = lax.axis_index("s")
    _stripes(s, lambda off, sz: pltpu.sync_copy(
        zeros.at[pl.ds(off, sz)], acc.at[pl.ds(off, sz)]))
    plsc.subcore_barrier()

    rows_per_core = epc // K
    rpt = rows_per_core // NS
    n_super = rpt // CH
    row0 = c * (ROWS_TOTAL - rows_per_core) + s * rpt

    @pl.when(c == 0)
    def _():
        _edge_loop(ya, esrc2, edst2, acc, srcv, dstv, rows, gsems, ssems,
                   isems, row0, n_super)

    @pl.when(c == 1)
    def _():
        _edge_loop(yb, esrc2, edst2, acc, srcv, dstv, rows, gsems, ssems,
                   isems, row0, n_super)

    plsc.subcore_barrier()
    _stripes(s, lambda off, sz: pltpu.sync_copy(
        acc.at[pl.ds(off, sz)], out.at[c, pl.ds(off, sz)]))


@functools.lru_cache
def _make_rows_pass(epc):
    return functools.partial(
        pl.kernel,
        functools.partial(_rows_body, epc),
        out_type=jax.ShapeDtypeStruct((NC, N, D), jnp.float32),
        mesh=_get_mesh(),
        scratch_types=[
            pltpu.VMEM((2, CH, K), jnp.int32),
            pltpu.VMEM((2, CH, K), jnp.int32),
            pltpu.VMEM((2, K, D), jnp.float32),
            pltpu.VMEM_SHARED((N, D), jnp.float32),
        ] + [pltpu.SemaphoreType.DMA] * 6,
    )()


def _deg_body(edst2, zeros, ones, out, dstv, onesv, acc, *ssems):
    c = lax.axis_index("c")
    s = lax.axis_index("s")
    _stripes(s, lambda off, sz: pltpu.sync_copy(
        zeros.at[pl.ds(off, sz)], acc.at[pl.ds(off, sz)]))
    pltpu.sync_copy(ones, onesv)
    plsc.subcore_barrier()

    rows_per_core = ROWS_TOTAL // NC
    rpt = rows_per_core // NS
    n_super = rpt // CH
    row0 = c * rows_per_core + s * rpt

    def drain():
        for b in range(CH):
            pltpu.make_async_copy(onesv, acc.at[dstv.at[b]], ssems[b]).wait()

    def body(t, carry):
        roff = row0 + t * CH

        @pl.when(t > 0)
        def _():
            drain()

        pltpu.sync_copy(edst2.at[pl.ds(roff, CH)], dstv)
        for b in range(CH):
            pltpu.async_copy(onesv, acc.at[dstv.at[b]], ssems[b], add=True)
        return carry

    lax.fori_loop(0, n_super, body, 0, unroll=False)
    drain()

    plsc.subcore_barrier()
    _stripes(s, lambda off, sz: pltpu.sync_copy(
        acc.at[pl.ds(off, sz)], out.at[c, pl.ds(off, sz)]))


@functools.lru_cache
def _get_deg_pass():
    return functools.partial(
        pl.kernel,
        _deg_body,
        out_type=jax.ShapeDtypeStruct((NC, N, D), jnp.float32),
        mesh=_get_mesh(),
        scratch_types=[
            pltpu.VMEM((CH, K), jnp.int32),
            pltpu.VMEM((K, D), jnp.float32),
            pltpu.VMEM_SHARED((N, D), jnp.float32),
        ] + [pltpu.SemaphoreType.DMA] * CH,
    )()


_row_spec = pl.BlockSpec((BK, D), lambda i: (i, 0))
_w_spec = pl.BlockSpec((D, D), lambda i: (0, 0))


def _mm(a, w):
    return jnp.dot(a, w, preferred_element_type=jnp.float32)


def _prep_body(dga, dgb, inp, h0, h1, u_o, yx0_o, yh0_o, yh1_o):
    u = lax.rsqrt(dga[...] + dgb[...] + 2.0)
    u_o[...] = u
    yx0_o[...] = u * inp[...]
    yh0_o[...] = u * h0[...]
    yh1_o[...] = u * h1[...]


def _prep(dga, dgb, inp, h0, h1):
    f32 = jnp.float32
    return pl.pallas_call(
        _prep_body,
        grid=(GRID,),
        in_specs=[_row_spec] * 5,
        out_specs=[_row_spec] * 4,
        out_shape=[jax.ShapeDtypeStruct((N, D), f32)] * 4,
    )(dga, dgb, inp, h0, h1)


def _stage1_body(nparts, *refs):
    refs = list(refs)
    tx = refs.pop(0)[...]
    if nparts == 2:
        tx = tx + refs.pop(0)[...]
    th, xin, h, u = (refs.pop(0)[...] for _ in range(4))
    wxz, whz, wxr, whr, wxh = (refs.pop(0)[...] for _ in range(5))
    z_o, gx_o, yrh_o = refs
    cx = u * (tx + 2.0 * u * xin)
    ch = u * (th + 2.0 * u * h)
    relu = lambda v: jnp.maximum(v, 0.0)
    z = jax.nn.sigmoid(relu(_mm(cx, wxz)) + relu(_mm(ch, whz)))
    r = jax.nn.sigmoid(relu(_mm(cx, wxr)) + relu(_mm(ch, whr)))
    z_o[...] = z
    gx_o[...] = relu(_mm(cx, wxh))
    yrh_o[...] = u * (r * h)


def _stage1(tx_parts, th, xin, h, u, wxz, whz, wxr, whr, wxh):
    f32 = jnp.float32
    nparts = len(tx_parts)
    return pl.pallas_call(
        functools.partial(_stage1_body, nparts),
        grid=(GRID,),
        in_specs=[_row_spec] * (nparts + 4) + [_w_spec] * 5,
        out_specs=[_row_spec] * 3,
        out_shape=[jax.ShapeDtypeStruct((N, D), f32)] * 3,
    )(*tx_parts, th, xin, h, u, wxz, whz, wxr, whr, wxh)


def _stage2_body(nparts, emit_ynext, *refs):
    refs = list(refs)
    trh = refs.pop(0)[...]
    if nparts == 2:
        trh = trh + refs.pop(0)[...]
    yrh, u, z, gx, h = (refs.pop(0)[...] for _ in range(5))
    whh = refs.pop(0)[...]
    crh = u * (trh + 2.0 * yrh)
    ht = jnp.tanh(gx + jnp.maximum(_mm(crh, whh), 0.0))
    hout = z * h + (1.0 - z) * ht
    refs[0][...] = hout
    if emit_ynext:
        refs[1][...] = u * hout


def _stage2(trh_parts, yrh, u, z, gx, h, whh, emit_ynext):
    f32 = jnp.float32
    nparts = len(trh_parts)
    n_out = 2 if emit_ynext else 1
    return pl.pallas_call(
        functools.partial(_stage2_body, nparts, emit_ynext),
        grid=(GRID,),
        in_specs=[_row_spec] * (nparts + 5) + [_w_spec],
        out_specs=[_row_spec] * n_out,
        out_shape=[jax.ShapeDtypeStruct((N, D), f32)] * n_out,
    )(*trh_parts, yrh, u, z, gx, h, whh)


def kernel(inp, edgidx, h, Wxz, Whz, Wxr, Whr, Wxh, Whh):
    f32 = jnp.float32
    esrc = edgidx[0].astype(jnp.int32).reshape(ROWS_TOTAL, K)
    edst = edgidx[1].astype(jnp.int32).reshape(ROWS_TOTAL, K)
    zeros = jnp.zeros((N, D), f32)
    ones = jnp.ones((K, D), f32)

    rows_paired = _make_rows_pass(E)
    rows_split = _make_rows_pass(E // 2)

    dg = _get_deg_pass()(edst, zeros, ones)
    u, yx0, yh0, yh1 = _prep(dg[0], dg[1], inp, h[0], h[1])

    p1 = rows_paired(yx0, yh0, esrc, edst, zeros)
    z0, gx0, yrh0 = _stage1([p1[0]], p1[1], inp, h[0], u,
                            Wxz[0], Whz[0], Wxr[0], Whr[0], Wxh[0])
    p2 = rows_paired(yrh0, yh1, esrc, edst, zeros)
    hout0, yx1 = _stage2([p2[0]], yrh0, u, z0, gx0, h[0], Whh[0], True)

    p3 = rows_split(yx1, yx1, esrc, edst, zeros)
    z1, gx1, yrh1 = _stage1([p3[0], p3[1]], p2[1], hout0, h[1], u,
                            Wxz[1], Whz[1], Wxr[1], Whr[1], Wxh[1])
    p4 = rows_split(yrh1, yrh1, esrc, edst, zeros)
    (hout1,) = _stage2([p4[0], p4[1]], yrh1, u, z1, gx1, h[1], Whh[1], False)

    h_out = jnp.stack([hout0, hout1], axis=0)
    return (h_out, h_out)
